# initial kernel scaffold (unmeasured)
import jax
import jax.numpy as jnp
from jax import lax
from jax.experimental import pallas as pl
from jax.experimental.pallas import tpu as pltpu


def kernel(
    x,
):
    def body(*refs):
        pass

    out_shape = jax.ShapeDtypeStruct(..., jnp.float32)
    return pl.pallas_call(body, out_shape=out_shape)(...)


# baseline (device time: 44866 ns/iter reference)
import jax
import jax.numpy as jnp
from jax import lax
from jax.experimental import pallas as pl
from jax.experimental.pallas import tpu as pltpu

N_Y = 4


def kernel(x):
    m_per, n = x.shape

    def body(x_ref, out_ref, comm_ref, send_sems, recv_sems):
        my_x = lax.axis_index("x")
        my_y = lax.axis_index("y")
        my_z = lax.axis_index("z")
        left = (my_y - 1) % N_Y
        right = (my_y + 1) % N_Y

        barrier_sem = pltpu.get_barrier_semaphore()
        for nbr in (left, right):
            pl.semaphore_signal(
                barrier_sem,
                inc=1,
                device_id=(my_x, nbr, my_z),
                device_id_type=pl.DeviceIdType.MESH,
            )
        pl.semaphore_wait(barrier_sem, 2)

        out_ref[pl.ds(my_y * m_per, m_per), :] = x_ref[...]
        comm_ref[0] = x_ref[...]

        for h in range(N_Y - 1):
            rdma = pltpu.make_async_remote_copy(
                src_ref=comm_ref.at[h],
                dst_ref=comm_ref.at[h + 1],
                send_sem=send_sems.at[h],
                recv_sem=recv_sems.at[h],
                device_id=(my_x, right, my_z),
                device_id_type=pl.DeviceIdType.MESH,
            )
            rdma.start()
            rdma.wait()
            origin = (my_y - h - 1) % N_Y
            out_ref[pl.ds(origin * m_per, m_per), :] = comm_ref[h + 1]

    return pl.pallas_call(
        body,
        out_shape=jax.ShapeDtypeStruct((N_Y * m_per, n), x.dtype),
        in_specs=[pl.BlockSpec(memory_space=pltpu.VMEM)],
        out_specs=pl.BlockSpec(memory_space=pltpu.VMEM),
        scratch_shapes=[
            pltpu.VMEM((N_Y, m_per, n), x.dtype),
            pltpu.SemaphoreType.DMA((N_Y - 1,)),
            pltpu.SemaphoreType.DMA((N_Y - 1,)),
        ],
        compiler_params=pltpu.CompilerParams(collective_id=0),
    )(x)


# device time: 42553 ns/iter; 1.0544x vs baseline; 1.0544x over previous
import jax
import jax.numpy as jnp
from jax import lax
from jax.experimental import pallas as pl
from jax.experimental.pallas import tpu as pltpu

N_Y = 4


def kernel(x):
    m_per, n = x.shape
    m2 = m_per // 2

    def body(x_ref, out_ref, full_ref, half_ref, send_sems, recv_sems):
        my_x = lax.axis_index("x")
        my_y = lax.axis_index("y")
        my_z = lax.axis_index("z")
        left_y = (my_y + N_Y - 1) % N_Y
        right_y = (my_y + 1) % N_Y
        opp_y = (my_y + 2) % N_Y
        left = (my_x, left_y, my_z)
        right = (my_x, right_y, my_z)

        barrier_sem = pltpu.get_barrier_semaphore()
        for nbr in (left, right):
            pl.semaphore_signal(
                barrier_sem, inc=1, device_id=nbr,
                device_id_type=pl.DeviceIdType.MESH,
            )
        pl.semaphore_wait(barrier_sem, 2)

        h1_right = pltpu.make_async_remote_copy(
            src_ref=x_ref,
            dst_ref=full_ref.at[0],
            send_sem=send_sems.at[0],
            recv_sem=recv_sems.at[0],
            device_id=right,
            device_id_type=pl.DeviceIdType.MESH,
        )
        h1_left = pltpu.make_async_remote_copy(
            src_ref=x_ref,
            dst_ref=full_ref.at[1],
            send_sem=send_sems.at[1],
            recv_sem=recv_sems.at[1],
            device_id=left,
            device_id_type=pl.DeviceIdType.MESH,
        )
        h1_right.start()
        h1_left.start()

        out_ref[pl.ds(my_y * m_per, m_per), :] = x_ref[...]

        h1_right.wait_recv()
        h2_right = pltpu.make_async_remote_copy(
            src_ref=full_ref.at[0, pl.ds(m2, m2), :],
            dst_ref=half_ref.at[0],
            send_sem=send_sems.at[2],
            recv_sem=recv_sems.at[2],
            device_id=right,
            device_id_type=pl.DeviceIdType.MESH,
        )
        h2_right.start()
        out_ref[pl.ds(left_y * m_per, m_per), :] = full_ref[0]

        h1_left.wait_recv()
        h2_left = pltpu.make_async_remote_copy(
            src_ref=full_ref.at[1, pl.ds(0, m2), :],
            dst_ref=half_ref.at[1],
            send_sem=send_sems.at[3],
            recv_sem=recv_sems.at[3],
            device_id=left,
            device_id_type=pl.DeviceIdType.MESH,
        )
        h2_left.start()
        out_ref[pl.ds(right_y * m_per, m_per), :] = full_ref[1]

        h2_left.wait_recv()
        out_ref[pl.ds(opp_y * m_per, m2), :] = half_ref[1]
        h2_right.wait_recv()
        out_ref[pl.ds(opp_y * m_per + m2, m2), :] = half_ref[0]

        h1_right.wait_send()
        h1_left.wait_send()
        h2_right.wait_send()
        h2_left.wait_send()

    return pl.pallas_call(
        body,
        out_shape=jax.ShapeDtypeStruct((N_Y * m_per, n), x.dtype),
        in_specs=[pl.BlockSpec(memory_space=pltpu.VMEM)],
        out_specs=pl.BlockSpec(memory_space=pltpu.VMEM),
        scratch_shapes=[
            pltpu.VMEM((2, m_per, n), x.dtype),
            pltpu.VMEM((2, m2, n), x.dtype),
            pltpu.SemaphoreType.DMA((4,)),
            pltpu.SemaphoreType.DMA((4,)),
        ],
        compiler_params=pltpu.CompilerParams(collective_id=0),
    )(x)


# device time: 36469 ns/iter; 1.2303x vs baseline; 1.1668x over previous
import jax
import jax.numpy as jnp
from jax import lax
from jax.experimental import pallas as pl
from jax.experimental.pallas import tpu as pltpu

N_Y = 4
S = N_Y - 1


def kernel(x):
    m_per, n = x.shape
    m2 = m_per // 2

    def body(
        x_ref, out_ref, own_buf, rs_buf, ls_buf, xr_buf, xl_buf,
        rs_ssem, rs_rsem, ls_ssem, ls_rsem,
        xr_ssem, xr_rsem, xl_ssem, xl_rsem,
    ):
        my_x = lax.axis_index("x")
        my_y = lax.axis_index("y")
        my_z = lax.axis_index("z")
        right = (my_x, jnp.minimum(my_y + 1, N_Y - 1), my_z)
        left = (my_x, jnp.maximum(my_y - 1, 0), my_z)
        peer = (1 - my_x, my_y, my_z)
        has_r = my_y < N_Y - 1
        has_l = my_y > 0
        edge = jnp.logical_or(my_y == 0, my_y == N_Y - 1)
        my_off = my_x * m2
        other_off = (1 - my_x) * m2

        def vs_rs(s):
            return jnp.logical_and(has_r, my_y >= s)

        def vr_rs(s):
            return my_y >= s + 1

        def vs_ls(s):
            return jnp.logical_and(has_l, my_y + s <= N_Y - 1)

        def vr_ls(s):
            return my_y + 1 + s <= N_Y - 1

        bar = pltpu.get_barrier_semaphore()

        @pl.when(has_r)
        def _():
            pl.semaphore_signal(
                bar, inc=1, device_id=right,
                device_id_type=pl.DeviceIdType.MESH,
            )

        @pl.when(has_l)
        def _():
            pl.semaphore_signal(
                bar, inc=1, device_id=left,
                device_id_type=pl.DeviceIdType.MESH,
            )

        pl.semaphore_signal(
            bar, inc=jnp.where(edge, 2, 1), device_id=peer,
            device_id_type=pl.DeviceIdType.MESH,
        )
        pl.semaphore_wait(bar, 3)

        own_buf[...] = x_ref[pl.ds(my_off, m2), :]

        def rcopy(src, dst, ssem, rsem, dev):
            return pltpu.make_async_remote_copy(
                src_ref=src, dst_ref=dst, send_sem=ssem, recv_sem=rsem,
                device_id=dev, device_id_type=pl.DeviceIdType.MESH,
            )

        rs_d = [
            rcopy(own_buf if s == 0 else rs_buf.at[s - 1], rs_buf.at[s],
                  rs_ssem.at[s], rs_rsem.at[s], right)
            for s in range(S)
        ]
        ls_d = [
            rcopy(own_buf if s == 0 else ls_buf.at[s - 1], ls_buf.at[s],
                  ls_ssem.at[s], ls_rsem.at[s], left)
            for s in range(S)
        ]
        xr_d = [
            rcopy(rs_buf.at[s], xr_buf.at[s], xr_ssem.at[s], xr_rsem.at[s],
                  peer)
            for s in range(S)
        ]
        xl_d = [
            rcopy(ls_buf.at[s], xl_buf.at[s], xl_ssem.at[s], xl_rsem.at[s],
                  peer)
            for s in range(S)
        ]

        @pl.when(vs_rs(0))
        def _():
            rs_d[0].start()

        @pl.when(vs_ls(0))
        def _():
            ls_d[0].start()

        out_ref[pl.ds(my_y * m_per, m_per), :] = x_ref[...]

        for s in range(S):
            @pl.when(vr_rs(s))
            def _(s=s):
                rs_d[s].wait_recv()
                xr_d[s].start()
                c = jnp.clip(my_y - 1 - s, 0, N_Y - 1)
                out_ref[pl.ds(c * m_per + my_off, m2), :] = rs_buf[s]

            if s + 1 < S:
                @pl.when(vs_rs(s + 1))
                def _(s=s):
                    rs_d[s + 1].start()

            @pl.when(vr_ls(s))
            def _(s=s):
                ls_d[s].wait_recv()
                xl_d[s].start()
                c = jnp.clip(my_y + 1 + s, 0, N_Y - 1)
                out_ref[pl.ds(c * m_per + my_off, m2), :] = ls_buf[s]

            if s + 1 < S:
                @pl.when(vs_ls(s + 1))
                def _(s=s):
                    ls_d[s + 1].start()

        for s in range(S):
            @pl.when(vr_rs(s))
            def _(s=s):
                xr_d[s].wait_recv()
                c = jnp.clip(my_y - 1 - s, 0, N_Y - 1)
                out_ref[pl.ds(c * m_per + other_off, m2), :] = xr_buf[s]

            @pl.when(vr_ls(s))
            def _(s=s):
                xl_d[s].wait_recv()
                c = jnp.clip(my_y + 1 + s, 0, N_Y - 1)
                out_ref[pl.ds(c * m_per + other_off, m2), :] = xl_buf[s]

        for s in range(S):
            @pl.when(vs_rs(s))
            def _(s=s):
                rs_d[s].wait_send()

            @pl.when(vs_ls(s))
            def _(s=s):
                ls_d[s].wait_send()

            @pl.when(vr_rs(s))
            def _(s=s):
                xr_d[s].wait_send()

            @pl.when(vr_ls(s))
            def _(s=s):
                xl_d[s].wait_send()

    return pl.pallas_call(
        body,
        out_shape=jax.ShapeDtypeStruct((N_Y * m_per, n), x.dtype),
        in_specs=[pl.BlockSpec(memory_space=pltpu.VMEM)],
        out_specs=pl.BlockSpec(memory_space=pltpu.VMEM),
        scratch_shapes=[
            pltpu.VMEM((m2, n), x.dtype),
            pltpu.VMEM((S, m2, n), x.dtype),
            pltpu.VMEM((S, m2, n), x.dtype),
            pltpu.VMEM((S, m2, n), x.dtype),
            pltpu.VMEM((S, m2, n), x.dtype),
            pltpu.SemaphoreType.DMA((S,)),
            pltpu.SemaphoreType.DMA((S,)),
            pltpu.SemaphoreType.DMA((S,)),
            pltpu.SemaphoreType.DMA((S,)),
            pltpu.SemaphoreType.DMA((S,)),
            pltpu.SemaphoreType.DMA((S,)),
            pltpu.SemaphoreType.DMA((S,)),
            pltpu.SemaphoreType.DMA((S,)),
        ],
        compiler_params=pltpu.CompilerParams(collective_id=0),
    )(x)


# device time: 29521 ns/iter; 1.5198x vs baseline; 1.2354x over previous
import jax
import jax.numpy as jnp
from jax import lax
from jax.experimental import pallas as pl
from jax.experimental.pallas import tpu as pltpu

N_Y = 4
S = N_Y - 1
Q = 4


def kernel(x):
    m_per, n = x.shape
    m2 = m_per // 2
    mq = m2 // Q

    def body(
        x_ref, out_ref, own_buf, rs_buf, ls_buf, xr_buf, xl_buf,
        rs_ssem, rs_rsem, ls_ssem, ls_rsem,
        xr_ssem, xr_rsem, xl_ssem, xl_rsem,
    ):
        my_x = lax.axis_index("x")
        my_y = lax.axis_index("y")
        my_z = lax.axis_index("z")
        right = (my_x, jnp.minimum(my_y + 1, N_Y - 1), my_z)
        left = (my_x, jnp.maximum(my_y - 1, 0), my_z)
        peer = (1 - my_x, my_y, my_z)
        has_r = my_y < N_Y - 1
        has_l = my_y > 0
        edge = jnp.logical_or(my_y == 0, my_y == N_Y - 1)
        my_off = my_x * m2
        other_off = (1 - my_x) * m2

        def vs_rs(s):
            return jnp.logical_and(has_r, my_y >= s)

        def vr_rs(s):
            return my_y >= s + 1

        def vs_ls(s):
            return jnp.logical_and(has_l, my_y + s <= N_Y - 1)

        def vr_ls(s):
            return my_y + 1 + s <= N_Y - 1

        bar = pltpu.get_barrier_semaphore()

        @pl.when(has_r)
        def _():
            pl.semaphore_signal(
                bar, inc=1, device_id=right,
                device_id_type=pl.DeviceIdType.MESH,
            )

        @pl.when(has_l)
        def _():
            pl.semaphore_signal(
                bar, inc=1, device_id=left,
                device_id_type=pl.DeviceIdType.MESH,
            )

        pl.semaphore_signal(
            bar, inc=jnp.where(edge, 2, 1), device_id=peer,
            device_id_type=pl.DeviceIdType.MESH,
        )
        pl.semaphore_wait(bar, 3)

        own_buf[...] = x_ref[pl.ds(my_off, m2), :]

        def rcopy(src, dst, ssem, rsem, dev):
            return pltpu.make_async_remote_copy(
                src_ref=src, dst_ref=dst, send_sem=ssem, recv_sem=rsem,
                device_id=dev, device_id_type=pl.DeviceIdType.MESH,
            )

        def sub(ref, q):
            return ref.at[pl.ds(q * mq, mq), :]

        rs_d = [
            [rcopy(sub(own_buf, q) if s == 0 else rs_buf.at[s - 1, q],
                   rs_buf.at[s, q],
                   rs_ssem.at[s * Q + q], rs_rsem.at[s * Q + q], right)
             for q in range(Q)]
            for s in range(S)
        ]
        ls_d = [
            [rcopy(sub(own_buf, q) if s == 0 else ls_buf.at[s - 1, q],
                   ls_buf.at[s, q],
                   ls_ssem.at[s * Q + q], ls_rsem.at[s * Q + q], left)
             for q in range(Q)]
            for s in range(S)
        ]
        xr_d = [
            [rcopy(rs_buf.at[s, q], xr_buf.at[s, q],
                   xr_ssem.at[s * Q + q], xr_rsem.at[s * Q + q], peer)
             for q in range(Q)]
            for s in range(S)
        ]
        xl_d = [
            [rcopy(ls_buf.at[s, q], xl_buf.at[s, q],
                   xl_ssem.at[s * Q + q], xl_rsem.at[s * Q + q], peer)
             for q in range(Q)]
            for s in range(S)
        ]

        for q in range(Q):
            @pl.when(vs_rs(0))
            def _(q=q):
                rs_d[0][q].start()

            @pl.when(vs_ls(0))
            def _(q=q):
                ls_d[0][q].start()

        out_ref[pl.ds(my_y * m_per, m_per), :] = x_ref[...]

        for s in range(S):
            for q in range(Q):
                @pl.when(vr_rs(s))
                def _(s=s, q=q):
                    rs_d[s][q].wait_recv()

                if s + 1 < S:
                    @pl.when(vs_rs(s + 1))
                    def _(s=s, q=q):
                        rs_d[s + 1][q].start()

                @pl.when(vr_rs(s))
                def _(s=s, q=q):
                    xr_d[s][q].start()
                    c = jnp.clip(my_y - 1 - s, 0, N_Y - 1)
                    out_ref[pl.ds(c * m_per + my_off + q * mq, mq), :] = (
                        rs_buf[s, q]
                    )

                @pl.when(vr_ls(s))
                def _(s=s, q=q):
                    ls_d[s][q].wait_recv()

                if s + 1 < S:
                    @pl.when(vs_ls(s + 1))
                    def _(s=s, q=q):
                        ls_d[s + 1][q].start()

                @pl.when(vr_ls(s))
                def _(s=s, q=q):
                    xl_d[s][q].start()
                    c = jnp.clip(my_y + 1 + s, 0, N_Y - 1)
                    out_ref[pl.ds(c * m_per + my_off + q * mq, mq), :] = (
                        ls_buf[s, q]
                    )

        for s in range(S):
            for q in range(Q):
                @pl.when(vr_rs(s))
                def _(s=s, q=q):
                    xr_d[s][q].wait_recv()
                    c = jnp.clip(my_y - 1 - s, 0, N_Y - 1)
                    out_ref[pl.ds(c * m_per + other_off + q * mq, mq), :] = (
                        xr_buf[s, q]
                    )

                @pl.when(vr_ls(s))
                def _(s=s, q=q):
                    xl_d[s][q].wait_recv()
                    c = jnp.clip(my_y + 1 + s, 0, N_Y - 1)
                    out_ref[pl.ds(c * m_per + other_off + q * mq, mq), :] = (
                        xl_buf[s, q]
                    )

        for s in range(S):
            for q in range(Q):
                @pl.when(vs_rs(s))
                def _(s=s, q=q):
                    rs_d[s][q].wait_send()

                @pl.when(vs_ls(s))
                def _(s=s, q=q):
                    ls_d[s][q].wait_send()

                @pl.when(vr_rs(s))
                def _(s=s, q=q):
                    xr_d[s][q].wait_send()

                @pl.when(vr_ls(s))
                def _(s=s, q=q):
                    xl_d[s][q].wait_send()

    return pl.pallas_call(
        body,
        out_shape=jax.ShapeDtypeStruct((N_Y * m_per, n), x.dtype),
        in_specs=[pl.BlockSpec(memory_space=pltpu.VMEM)],
        out_specs=pl.BlockSpec(memory_space=pltpu.VMEM),
        scratch_shapes=[
            pltpu.VMEM((m2, n), x.dtype),
            pltpu.VMEM((S, Q, mq, n), x.dtype),
            pltpu.VMEM((S, Q, mq, n), x.dtype),
            pltpu.VMEM((S, Q, mq, n), x.dtype),
            pltpu.VMEM((S, Q, mq, n), x.dtype),
            pltpu.SemaphoreType.DMA((S * Q,)),
            pltpu.SemaphoreType.DMA((S * Q,)),
            pltpu.SemaphoreType.DMA((S * Q,)),
            pltpu.SemaphoreType.DMA((S * Q,)),
            pltpu.SemaphoreType.DMA((S * Q,)),
            pltpu.SemaphoreType.DMA((S * Q,)),
            pltpu.SemaphoreType.DMA((S * Q,)),
            pltpu.SemaphoreType.DMA((S * Q,)),
        ],
        compiler_params=pltpu.CompilerParams(collective_id=0),
    )(x)
